# initial kernel scaffold (unmeasured)
import jax
import jax.numpy as jnp
from jax import lax
from jax.experimental import pallas as pl
from jax.experimental.pallas import tpu as pltpu

N_DEV = 16
BLK_M = 256
K = 4096
N = 2048

WIRE_DTYPE = jnp.float8_e5m2
DOT_DTYPE = jnp.bfloat16


def kernel(x, w_mat, scale_x, scale_w):
    def body(x_ref, w_ref, sx_ref, sw_ref, out_ref,
             xsend, recv_buf, acc_ref, send_sems, recv_sems):
        my = lax.axis_index("i")

        barrier_sem = pltpu.get_barrier_semaphore()
        for d in range(1, N_DEV):
            peer = lax.rem(my + d, N_DEV)
            pl.semaphore_signal(
                barrier_sem, inc=1,
                device_id=(peer,), device_id_type=pl.DeviceIdType.MESH,
            )
        pl.semaphore_wait(barrier_sem, N_DEV - 1)

        sends = []
        for d in range(1, N_DEV):
            peer = lax.rem(my + d, N_DEV)
            xsend[d] = x_ref[pl.ds(peer * BLK_M, BLK_M), :].astype(WIRE_DTYPE)
            rdma = pltpu.make_async_remote_copy(
                src_ref=xsend.at[d],
                dst_ref=recv_buf.at[N_DEV - d],
                send_sem=send_sems.at[d],
                recv_sem=recv_sems.at[N_DEV - d],
                device_id=(peer,),
                device_id_type=pl.DeviceIdType.MESH,
            )
            rdma.start()
            sends.append(rdma)

        own = x_ref[pl.ds(my * BLK_M, BLK_M), :].astype(DOT_DTYPE)
        w_own = w_ref[pl.ds(my * BLK_M, BLK_M), :].astype(DOT_DTYPE)
        acc_ref[...] = jax.lax.dot_general(
            own, w_own, (((1,), (0,)), ((), ())),
            preferred_element_type=jnp.float32,
        )

        for k in range(1, N_DEV):
            recv = pltpu.make_async_remote_copy(
                src_ref=recv_buf.at[k],
                dst_ref=recv_buf.at[k],
                send_sem=send_sems.at[k],
                recv_sem=recv_sems.at[k],
                device_id=(my,),
                device_id_type=pl.DeviceIdType.MESH,
            )
            recv.wait_recv()
            s = lax.rem(my + k, N_DEV)
            blk = recv_buf[k].astype(DOT_DTYPE)
            w_blk = w_ref[pl.ds(s * BLK_M, BLK_M), :].astype(DOT_DTYPE)
            acc_ref[...] += jax.lax.dot_general(
                blk, w_blk, (((1,), (0,)), ((), ())),
                preferred_element_type=jnp.float32,
            )

        y = acc_ref[...] * (sx_ref[0] * sw_ref[0])
        z = jnp.clip(y, -60.0, 60.0)
        out_ref[...] = y / (1.0 + jnp.exp(-z))

        for rdma in sends:
            rdma.wait_send()

    m_total, k = x.shape
    return pl.pallas_call(
        body,
        out_shape=jax.ShapeDtypeStruct((BLK_M, N), jnp.float32),
        in_specs=[
            pl.BlockSpec(memory_space=pltpu.VMEM),
            pl.BlockSpec(memory_space=pltpu.VMEM),
            pl.BlockSpec(memory_space=pltpu.SMEM),
            pl.BlockSpec(memory_space=pltpu.SMEM),
        ],
        out_specs=pl.BlockSpec(memory_space=pltpu.VMEM),
        scratch_shapes=[
            pltpu.VMEM((N_DEV, BLK_M, k), WIRE_DTYPE),
            pltpu.VMEM((N_DEV, BLK_M, k), WIRE_DTYPE),
            pltpu.VMEM((BLK_M, N), jnp.float32),
            pltpu.SemaphoreType.DMA((N_DEV,)),
            pltpu.SemaphoreType.DMA((N_DEV,)),
        ],
        compiler_params=pltpu.CompilerParams(collective_id=0),
    )(x, w_mat, scale_x, scale_w)


# baseline (device time: 36554 ns/iter reference)
import jax
import jax.numpy as jnp
from jax import lax
from jax.experimental import pallas as pl
from jax.experimental.pallas import tpu as pltpu

N_DEV = 16
BLK_M = 256
K = 4096
N = 2048

WIRE_DTYPE = jnp.float8_e5m2
DOT_DTYPE = jnp.bfloat16


def kernel(x, w_mat, scale_x, scale_w):
    def body(x_ref, w_ref, sx_ref, sw_ref, out_ref,
             xsend, recv_buf, acc_ref, send_sems, recv_sems):
        my = lax.axis_index("i")

        barrier_sem = pltpu.get_barrier_semaphore()
        for d in range(1, N_DEV):
            peer = lax.rem(my + d, N_DEV)
            pl.semaphore_signal(
                barrier_sem, inc=1,
                device_id=(peer,), device_id_type=pl.DeviceIdType.MESH,
            )
        pl.semaphore_wait(barrier_sem, N_DEV - 1)

        sends = []
        for d in range(1, N_DEV):
            peer = lax.rem(my + d, N_DEV)
            xsend[d] = x_ref[pl.ds(peer * BLK_M, BLK_M), :].astype(WIRE_DTYPE)
            rdma = pltpu.make_async_remote_copy(
                src_ref=xsend.at[d],
                dst_ref=recv_buf.at[N_DEV - d],
                send_sem=send_sems.at[d],
                recv_sem=recv_sems.at[N_DEV - d],
                device_id=(peer,),
                device_id_type=pl.DeviceIdType.MESH,
            )
            rdma.start()
            sends.append(rdma)

        own = x_ref[pl.ds(my * BLK_M, BLK_M), :].astype(DOT_DTYPE)
        w_own = w_ref[pl.ds(my * BLK_M, BLK_M), :].astype(DOT_DTYPE)
        acc_ref[...] = jax.lax.dot_general(
            own, w_own, (((1,), (0,)), ((), ())),
            preferred_element_type=jnp.float32,
        )

        for k in range(1, N_DEV):
            recv = pltpu.make_async_remote_copy(
                src_ref=recv_buf.at[k],
                dst_ref=recv_buf.at[k],
                send_sem=send_sems.at[k],
                recv_sem=recv_sems.at[k],
                device_id=(my,),
                device_id_type=pl.DeviceIdType.MESH,
            )
            recv.wait_recv()
            s = lax.rem(my + k, N_DEV)
            blk = recv_buf[k].astype(DOT_DTYPE)
            w_blk = w_ref[pl.ds(s * BLK_M, BLK_M), :].astype(DOT_DTYPE)
            acc_ref[...] += jax.lax.dot_general(
                blk, w_blk, (((1,), (0,)), ((), ())),
                preferred_element_type=jnp.float32,
            )

        y = acc_ref[...] * (sx_ref[0] * sw_ref[0])
        z = jnp.clip(y, -60.0, 60.0)
        out_ref[...] = y / (1.0 + jnp.exp(-z))

        for rdma in sends:
            rdma.wait_send()

    m_total, k = x.shape
    return pl.pallas_call(
        body,
        out_shape=jax.ShapeDtypeStruct((BLK_M, N), jnp.float32),
        in_specs=[
            pl.BlockSpec(memory_space=pltpu.VMEM),
            pl.BlockSpec(memory_space=pltpu.VMEM),
            pl.BlockSpec(memory_space=pltpu.SMEM),
            pl.BlockSpec(memory_space=pltpu.SMEM),
        ],
        out_specs=pl.BlockSpec(memory_space=pltpu.VMEM),
        scratch_shapes=[
            pltpu.VMEM((N_DEV, BLK_M, k), WIRE_DTYPE),
            pltpu.VMEM((N_DEV, BLK_M, k), WIRE_DTYPE),
            pltpu.VMEM((BLK_M, N), jnp.float32),
            pltpu.SemaphoreType.DMA((N_DEV,)),
            pltpu.SemaphoreType.DMA((N_DEV,)),
        ],
        compiler_params=pltpu.CompilerParams(
            collective_id=0,
            vmem_limit_bytes=100 * 1024 * 1024,
        ),
    )(x, w_mat, scale_x, scale_w)


# device time: 32940 ns/iter; 1.1097x vs baseline; 1.1097x over previous
import jax
import jax.numpy as jnp
from jax import lax
from jax.experimental import pallas as pl
from jax.experimental.pallas import tpu as pltpu

N_DEV = 16
BLK_M = 256
K = 4096
N = 2048

WIRE_DTYPE = jnp.float8_e5m2
DOT_DTYPE = jnp.float8_e5m2


def kernel(x, w_mat, scale_x, scale_w):
    def body(x_ref, w_hbm, sx_ref, sw_ref, out_ref,
             xsend, recv_buf, wbuf, acc_ref, send_sems, recv_sems, wsems):
        my = lax.axis_index("i")

        def w_copy(k, slot):
            s = lax.rem(my + k, N_DEV)
            return pltpu.make_async_copy(
                w_hbm.at[pl.ds(s * BLK_M, BLK_M), :],
                wbuf.at[slot],
                wsems.at[slot],
            )

        w_copy(0, 0).start()
        w_copy(1, 1).start()

        barrier_sem = pltpu.get_barrier_semaphore()
        for d in range(1, N_DEV):
            peer = lax.rem(my + d, N_DEV)
            pl.semaphore_signal(
                barrier_sem, inc=1,
                device_id=(peer,), device_id_type=pl.DeviceIdType.MESH,
            )
        pl.semaphore_wait(barrier_sem, N_DEV - 1)

        sends = []
        for d in range(1, N_DEV):
            peer = lax.rem(my + d, N_DEV)
            xsend[d, :, :] = x_ref[pl.ds(peer * BLK_M, BLK_M), :].astype(WIRE_DTYPE)
            rdma = pltpu.make_async_remote_copy(
                src_ref=xsend.at[d],
                dst_ref=recv_buf.at[N_DEV - d],
                send_sem=send_sems.at[d],
                recv_sem=recv_sems.at[N_DEV - d],
                device_id=(peer,),
                device_id_type=pl.DeviceIdType.MESH,
            )
            rdma.start()
            sends.append(rdma)

        xsend[0, :, :] = x_ref[pl.ds(my * BLK_M, BLK_M), :].astype(WIRE_DTYPE)

        for k in range(N_DEV):
            w_copy(k, k % 2).wait()
            if k >= 1:
                recv = pltpu.make_async_remote_copy(
                    src_ref=recv_buf.at[k],
                    dst_ref=recv_buf.at[k],
                    send_sem=send_sems.at[k],
                    recv_sem=recv_sems.at[k],
                    device_id=(my,),
                    device_id_type=pl.DeviceIdType.MESH,
                )
                recv.wait_recv()
                xblk = recv_buf[k, :, :]
            else:
                xblk = xsend[0, :, :]
            w_blk = wbuf[k % 2].astype(DOT_DTYPE)
            term = jax.lax.dot_general(
                xblk, w_blk, (((1,), (0,)), ((), ())),
                preferred_element_type=jnp.float32,
            )
            if k == 0:
                acc_ref[...] = term
            else:
                acc_ref[...] += term
            if k + 2 < N_DEV:
                w_copy(k + 2, k % 2).start()

        y = acc_ref[...] * (sx_ref[0] * sw_ref[0])
        z = jnp.clip(y, -60.0, 60.0)
        out_ref[...] = y / (1.0 + jnp.exp(-z))

        for rdma in sends:
            rdma.wait_send()

    m_total, k = x.shape
    return pl.pallas_call(
        body,
        out_shape=jax.ShapeDtypeStruct((BLK_M, N), jnp.float32),
        in_specs=[
            pl.BlockSpec(memory_space=pltpu.VMEM),
            pl.BlockSpec(memory_space=pltpu.MemorySpace.HBM),
            pl.BlockSpec(memory_space=pltpu.SMEM),
            pl.BlockSpec(memory_space=pltpu.SMEM),
        ],
        out_specs=pl.BlockSpec(memory_space=pltpu.VMEM),
        scratch_shapes=[
            pltpu.VMEM((N_DEV, BLK_M, k), WIRE_DTYPE),
            pltpu.VMEM((N_DEV, BLK_M, k), WIRE_DTYPE),
            pltpu.VMEM((2, BLK_M, N), jnp.float32),
            pltpu.VMEM((BLK_M, N), jnp.float32),
            pltpu.SemaphoreType.DMA((N_DEV,)),
            pltpu.SemaphoreType.DMA((N_DEV,)),
            pltpu.SemaphoreType.DMA((2,)),
        ],
        compiler_params=pltpu.CompilerParams(
            collective_id=0,
            vmem_limit_bytes=64 * 1024 * 1024,
        ),
    )(x, w_mat, scale_x, scale_w)


# device time: 21281 ns/iter; 1.7177x vs baseline; 1.5479x over previous
import os

import jax
import jax.numpy as jnp
from jax import lax
from jax.experimental import pallas as pl
from jax.experimental.pallas import tpu as pltpu

_KEXP = os.environ.get("KEXP", "")

N_DEV = 16
BLK_M = 256
K = 4096
N = 2048

WIRE_DTYPE = jnp.float8_e5m2


def kernel(x, w_mat, scale_x, scale_w):
    def body(x_ref, w_hbm, sx_ref, sw_ref, out_ref,
             xsend, xg, w8, wstage, send_sems, recv_sems, wsems):
        my = lax.axis_index("i")

        def w_copy(k, slot):
            s = lax.rem(my + k, N_DEV)
            return pltpu.make_async_copy(
                w_hbm.at[pl.ds(s * BLK_M, BLK_M), :],
                wstage.at[slot],
                wsems.at[slot],
            )

        w_copy(0, 0).start()
        w_copy(1, 1).start()

        if _KEXP != "nocomm":
            barrier_sem = pltpu.get_barrier_semaphore()
            for d in range(1, N_DEV):
                peer = lax.rem(my + d, N_DEV)
                pl.semaphore_signal(
                    barrier_sem, inc=1,
                    device_id=(peer,), device_id_type=pl.DeviceIdType.MESH,
                )
            pl.semaphore_wait(barrier_sem, N_DEV - 1)

        sends = []
        for d in range(1, N_DEV):
            peer = lax.rem(my + d, N_DEV)
            xsend[d, :, :] = x_ref[pl.ds(peer * BLK_M, BLK_M), :].astype(WIRE_DTYPE)
            if _KEXP == "nocomm":
                continue
            kslot = N_DEV - d
            rdma = pltpu.make_async_remote_copy(
                src_ref=xsend.at[d],
                dst_ref=xg.at[:, pl.ds(kslot * BLK_M, BLK_M)],
                send_sem=send_sems.at[d],
                recv_sem=recv_sems.at[kslot],
                device_id=(peer,),
                device_id_type=pl.DeviceIdType.MESH,
            )
            rdma.start()
            sends.append(rdma)

        xg[:, 0:BLK_M] = x_ref[pl.ds(my * BLK_M, BLK_M), :].astype(WIRE_DTYPE)

        for k in range(N_DEV):
            w_copy(k, k % 2).wait()
            if k + 2 < N_DEV:
                pass
            if k >= 1 and _KEXP != "nocomm":
                recv = pltpu.make_async_remote_copy(
                    src_ref=xsend.at[k],
                    dst_ref=xg.at[:, pl.ds(k * BLK_M, BLK_M)],
                    send_sem=send_sems.at[k],
                    recv_sem=recv_sems.at[k],
                    device_id=(my,),
                    device_id_type=pl.DeviceIdType.MESH,
                )
                recv.wait_recv()
            w8[pl.ds(k * BLK_M, BLK_M), :] = wstage[k % 2].astype(WIRE_DTYPE)
            if k + 2 < N_DEV:
                w_copy(k + 2, k % 2).start()

        if _KEXP != "nocomp":
            acc = jax.lax.dot_general(
                xg[...], w8[...], (((1,), (0,)), ((), ())),
                preferred_element_type=jnp.float32,
            )
        else:
            acc = jax.lax.dot_general(
                xg[:, 0:BLK_M], w8[0:BLK_M, :], (((1,), (0,)), ((), ())),
                preferred_element_type=jnp.float32,
            )

        y = acc * (sx_ref[0] * sw_ref[0])
        z = jnp.clip(y, -60.0, 60.0)
        out_ref[...] = y / (1.0 + jnp.exp(-z))

        for rdma in sends:
            rdma.wait_send()

    m_total, kk = x.shape
    return pl.pallas_call(
        body,
        out_shape=jax.ShapeDtypeStruct((BLK_M, N), jnp.float32),
        in_specs=[
            pl.BlockSpec(memory_space=pltpu.VMEM),
            pl.BlockSpec(memory_space=pltpu.MemorySpace.HBM),
            pl.BlockSpec(memory_space=pltpu.SMEM),
            pl.BlockSpec(memory_space=pltpu.SMEM),
        ],
        out_specs=pl.BlockSpec(memory_space=pltpu.VMEM),
        scratch_shapes=[
            pltpu.VMEM((N_DEV, BLK_M, kk), WIRE_DTYPE),
            pltpu.VMEM((BLK_M, K), WIRE_DTYPE),
            pltpu.VMEM((K, N), WIRE_DTYPE),
            pltpu.VMEM((2, BLK_M, N), jnp.float32),
            pltpu.SemaphoreType.DMA((N_DEV,)),
            pltpu.SemaphoreType.DMA((N_DEV,)),
            pltpu.SemaphoreType.DMA((2,)),
        ],
        compiler_params=pltpu.CompilerParams(
            collective_id=None if _KEXP == "nocomm" else 0,
            vmem_limit_bytes=64 * 1024 * 1024,
        ),
    )(x, w_mat, scale_x, scale_w)
